# baseline (speedup 1.0000x reference)
import functools

import jax
import jax.numpy as jnp
from jax import lax
from jax.experimental import pallas as pl
from jax.experimental.pallas import tpu as pltpu
from jax.experimental.pallas import tpu_sc as plsc

C = 16
G = 128
V = G * G * G
N = 1048576

_NC = 2
_NS = 16
_NW = _NC * _NS


_BS = 131072


def _snap(t):
    w = t.astype(jnp.int32)
    f = t - w.astype(jnp.float32)
    up = (f > 0.5) | ((f == 0.5) & ((w & 1) == 1))
    return w + jnp.where(up, 1, 0)


def _idx_body(x_ref, y_ref, z_ref, o_ref):
    ix = _snap(x_ref[...] * (G - 1.0))
    iy = _snap(y_ref[...] * (G - 1.0))
    iz = _snap(z_ref[...] * (G - 1.0))
    o_ref[...] = (ix * G + iy) * G + iz


def _compute_indices(cx, cy, cz):
    spec = pl.BlockSpec((_BS,), lambda i: (i,))
    return pl.pallas_call(
        _idx_body,
        grid=(N // _BS,),
        in_specs=[spec, spec, spec],
        out_specs=spec,
        out_shape=jax.ShapeDtypeStruct((N,), jnp.int32),
    )(cx, cy, cz)


_TS = 1024
_CPW = V // _NW
_NSL = _CPW // _TS

_SC_PARAMS = pltpu.CompilerParams(
    use_tc_tiling_on_sc=False, needs_layout_passes=False)


def _make_table(grid16):
    mesh = plsc.VectorSubcoreMesh(core_axis_name="c", subcore_axis_name="s")

    @functools.partial(
        pl.kernel,
        mesh=mesh,
        compiler_params=_SC_PARAMS,
        out_type=jax.ShapeDtypeStruct((V * C,), jnp.float32),
        scratch_types=[
            pltpu.VMEM((2, C, _TS), jnp.float32),
            pltpu.VMEM((2, _TS * C), jnp.float32),
            pltpu.SemaphoreType.DMA,
            pltpu.SemaphoreType.DMA,
            pltpu.SemaphoreType.DMA,
            pltpu.SemaphoreType.DMA,
        ],
    )
    def body(g_hbm, t_hbm, in_v, out_v, is0, is1, os0, os1):
        wid = lax.axis_index("s") * _NC + lax.axis_index("c")
        base = wid * _CPW
        isem = (is0, is1)
        osem = (os0, os1)
        lane = lax.iota(jnp.int32, 16)

        def start_in(s, p):
            off = base + s * _TS
            for r in range(C):
                pltpu.async_copy(
                    g_hbm.at[r, pl.ds(off, _TS)], in_v.at[p, r], isem[p])

        def wait_in(s, p):
            off = base + s * _TS
            for r in range(C):
                pltpu.make_async_copy(
                    g_hbm.at[r, pl.ds(off, _TS)], in_v.at[p, r],
                    isem[p]).wait()

        def transpose(p):
            def step(o, carry):
                tbase = (o * 16 + lane) * 16
                for r in range(C):
                    val = in_v[p, r, pl.ds(o * 16, 16)]
                    plsc.store_scatter(out_v.at[p], [tbase + r], val)
                return carry
            lax.fori_loop(0, _TS // 16, step, 0)

        def start_out(s, p):
            off = (base + s * _TS) * C
            pltpu.async_copy(
                out_v.at[p], t_hbm.at[pl.ds(off, _TS * C)], osem[p])

        def wait_out(s, p):
            off = (base + s * _TS) * C
            pltpu.make_async_copy(
                out_v.at[p], t_hbm.at[pl.ds(off, _TS * C)], osem[p]).wait()

        start_in(0, 0)

        def pair(i, carry):
            s = i * 2
            wait_in(s, 0)
            start_in(s + 1, 1)

            @pl.when(i > 0)
            def _():
                wait_out(s - 2, 0)
            transpose(0)

            @pl.when(i > 0)
            def _():
                start_out(s - 1, 1)
            wait_in(s + 1, 1)

            @pl.when(i + 1 < _NSL // 2)
            def _():
                start_in(s + 2, 0)

            @pl.when(i > 0)
            def _():
                wait_out(s - 1, 1)
            transpose(1)
            start_out(s, 0)
            return carry

        lax.fori_loop(0, _NSL // 2, pair, 0)
        wait_out(_NSL - 2, 0)
        plsc.subcore_barrier()
        start_out(_NSL - 1, 1)
        wait_out(_NSL - 1, 1)

    return body(grid16)


_BPW = N // _NW
_CH = 1024
_NCH = _BPW // _CH
_IPG = 128
_GPC = _CH // _IPG
_NB = N // 128


def _gather_all(idx, table):
    mesh = plsc.VectorSubcoreMesh(core_axis_name="c", subcore_axis_name="s")

    @functools.partial(
        pl.kernel,
        mesh=mesh,
        compiler_params=_SC_PARAMS,
        out_type=jax.ShapeDtypeStruct((2 * _NB * 8 * 128,), jnp.float32),
        scratch_types=[
            pltpu.VMEM((4, _CH), jnp.int32),
            pltpu.VMEM((3, _CH, C), jnp.float32),
            pltpu.VMEM((2, 2 * (_CH // 128) * 8 * 128), jnp.float32),
            pltpu.SemaphoreType.DMA,
            pltpu.SemaphoreType.DMA,
            pltpu.SemaphoreType.DMA,
            pltpu.SemaphoreType.DMA,
            pltpu.SemaphoreType.DMA,
            pltpu.SemaphoreType.DMA,
            pltpu.SemaphoreType.DMA,
            pltpu.SemaphoreType.DMA,
            pltpu.SemaphoreType.DMA,
        ],
    )
    def body(idx_hbm, table_hbm, out_hbm,
             idx_v, rows_v, tr_v, cs0, cs1, cs2, cs3, gs0, gs1, gs2,
             ws0, ws1):
        wid = lax.axis_index("s") * _NC + lax.axis_index("c")
        base = wid * _BPW
        csem = (cs0, cs1, cs2, cs3)
        gsem = (gs0, gs1, gs2)
        wsem = (ws0, ws1)
        lane = lax.iota(jnp.int32, 16)

        def copy_idx(g):
            q = g & 3
            off = base + g * _CH
            return pltpu.async_copy(
                idx_hbm.at[pl.ds(off, _CH)], idx_v.at[q], csem[q])

        def fire_gathers(g, p):
            q = g & 3
            return [
                pltpu.async_copy(
                    table_hbm.at[idx_v.at[q, pl.ds(j * _IPG, _IPG)]],
                    rows_v.at[p, pl.ds(j * _IPG, _IPG)],
                    gsem[p])
                for j in range(_GPC)
            ]

        cpat = ((lane >> 3) * ((_CH // 128) * 1024)) + ((lane & 7) * 128)

        def transpose(p3, p2):
            def step(o, carry):
                pt0 = o * 16
                for u in range(16):
                    pt = pt0 + u
                    val = rows_v[p3, pt, :]
                    dst = cpat + ((pt >> 7) * 1024 + (pt & 127))
                    plsc.store_scatter(tr_v.at[p2], [dst], val)
                return carry
            lax.fori_loop(0, _CH // 16, step, 0)

        _HALF = (_CH // 128) * 1024

        def writeback(g, p):
            blk0 = (base + g * _CH) // 128
            return [
                pltpu.async_copy(
                    tr_v.at[p, pl.ds(h * _HALF, _HALF)],
                    out_hbm.at[pl.ds(h * _NB * 1024 + blk0 * 1024, _HALF)],
                    wsem[p])
                for h in range(2)
            ]

        gh = {}
        wb = [None, None]
        ch = {0: copy_idx(0), 1: copy_idx(1)}
        ch[0].wait()

        def drain_and_emit(t):
            for h in gh.pop(t):
                h.wait()
            p2 = t & 1
            if wb[p2] is not None:
                for h in wb[p2]:
                    h.wait()
            transpose3(t)
            if t >= 1:
                wb[1 - p2] = writeback(t - 1, 1 - p2)

        def transpose3(t):
            transpose(t % 3, t & 1)

        for g in range(_NCH):
            gh[g] = fire_gathers(g, g % 3)
            if g >= 2:
                drain_and_emit(g - 2)
            if g + 2 < _NCH:
                ch[g + 2] = copy_idx(g + 2)
            if g + 1 < _NCH:
                ch[g + 1].wait()
        drain_and_emit(_NCH - 2)
        for h in gh.pop(_NCH - 1):
            h.wait()
        p2 = (_NCH - 1) & 1
        if wb[p2] is not None:
            for h in wb[p2]:
                h.wait()
        transpose3(_NCH - 1)
        wb[1 - p2] = writeback(_NCH - 2, 1 - p2)
        plsc.subcore_barrier()
        wb[p2] = writeback(_NCH - 1, p2)
        for h in wb[0] + wb[1]:
            h.wait()

    return body(idx, table)


def kernel(coords, grid):
    table = _make_table(grid.reshape(C, V)).reshape(V, C)
    idx = _compute_indices(coords[:, 0], coords[:, 1], coords[:, 2])
    res = _gather_all(idx, table)
    return res.reshape(2, _NB, 8, 128).transpose(1, 3, 0, 2).reshape(N, C)
